# Initial kernel scaffold; baseline (speedup 1.0000x reference)
import jax
import jax.numpy as jnp
from jax.experimental import pallas as pl


def kernel(x, edge_index, W1, b1, W2, b2, W3, b3, W4, b4):
    raise NotImplementedError("write your pallas kernel here")


# speedup vs baseline: 6.1027x; 6.1027x over previous
import functools

import jax
import jax.numpy as jnp
from jax import lax
from jax.experimental import pallas as pl
from jax.experimental.pallas import tpu as pltpu
from jax.experimental.pallas import tpu_sc as plsc

NC = 2
NS = 16
CHUNK = 128
ROW_BLOCK = 2000


def _sc_mesh():
  return plsc.VectorSubcoreMesh(
      core_axis_name="c", subcore_axis_name="s", num_cores=NC,
      num_subcores=NS)


def _zero_fill(ref, width):
  zeros = jnp.zeros((16,), jnp.float32)

  def body(i, _):
    for w in range(width // 16):
      ref[i, pl.ds(w * 16, 16)] = zeros
    return 0

  lax.fori_loop(0, CHUNK, body, 0)


def _ones_fill(ref):
  ones = jnp.ones((16,), jnp.float32)

  def body(i, _):
    ref[i] = ones
    return 0

  lax.fori_loop(0, CHUNK, body, 0)


def _zero_slab(zero_v, acc_s, row0, rows):
  full, rem = divmod(rows, CHUNK)
  for i in range(full):
    pltpu.sync_copy(zero_v, acc_s.at[pl.ds(row0 + i * CHUNK, CHUNK)])
  if rem:
    pltpu.sync_copy(zero_v.at[pl.ds(0, rem)],
                    acc_s.at[pl.ds(row0 + full * CHUNK, rem)])


def _make_sc_deg(nrows, chunks_per_tile):
  @functools.partial(
      pl.kernel,
      out_type=(jax.ShapeDtypeStruct((nrows, 16), jnp.float32),
                jax.ShapeDtypeStruct((nrows, 16), jnp.float32)),
      mesh=_sc_mesh(),
      scratch_types=[
          pltpu.VMEM((chunks_per_tile, CHUNK), jnp.int32),
          pltpu.VMEM((CHUNK, 16), jnp.float32),
          pltpu.VMEM((CHUNK, 16), jnp.float32),
          pltpu.VMEM_SHARED((nrows, 16), jnp.float32),
      ],
  )
  def deg_kernel(dst_hbm, out0, out1, dst_v, ones_v, zero_v, acc_s):
    cid = lax.axis_index("c")
    sid = lax.axis_index("s")
    wid = cid * NS + sid
    slab = nrows // NS

    pltpu.sync_copy(dst_hbm.at[pl.ds(wid * chunks_per_tile, chunks_per_tile)],
                    dst_v)
    _ones_fill(ones_v)
    _zero_fill(zero_v, 16)
    _zero_slab(zero_v, acc_s, sid * slab, slab)
    plsc.subcore_barrier()

    def body(j, _):
      pltpu.sync_copy(ones_v, acc_s.at[dst_v.at[j]], add=True)
      return 0

    lax.fori_loop(0, chunks_per_tile, body, 0)
    plsc.subcore_barrier()

    @pl.when(cid == 0)
    def _():
      pltpu.sync_copy(acc_s.at[pl.ds(sid * slab, slab)],
                      out0.at[pl.ds(sid * slab, slab)])

    @pl.when(cid == 1)
    def _():
      pltpu.sync_copy(acc_s.at[pl.ds(sid * slab, slab)],
                      out1.at[pl.ds(sid * slab, slab)])

  return deg_kernel


def _make_sc_agg(nrows, chunks_per_tile, width):
  @functools.partial(
      pl.kernel,
      out_type=(jax.ShapeDtypeStruct((nrows, width), jnp.float32),
                jax.ShapeDtypeStruct((nrows, width), jnp.float32)),
      mesh=_sc_mesh(),
      scratch_types=[
          pltpu.VMEM((chunks_per_tile, CHUNK), jnp.int32),
          pltpu.VMEM((chunks_per_tile, CHUNK), jnp.int32),
          pltpu.VMEM((CHUNK, width), jnp.float32),
          pltpu.VMEM_SHARED((nrows, width), jnp.float32),
          pltpu.SemaphoreType.DMA,
      ],
  )
  def agg_kernel(src_hbm, dst_hbm, glo_hbm, ghi_hbm, out_lo, out_hi,
                 src_v, dst_v, rows_v, acc_s, sem):
    cid = lax.axis_index("c")
    sid = lax.axis_index("s")
    slab = nrows // NS

    pltpu.sync_copy(src_hbm.at[pl.ds(sid * chunks_per_tile, chunks_per_tile)],
                    src_v)
    pltpu.sync_copy(dst_hbm.at[pl.ds(sid * chunks_per_tile, chunks_per_tile)],
                    dst_v)
    _zero_fill(rows_v, width)
    _zero_slab(rows_v, acc_s, sid * slab, slab)
    plsc.subcore_barrier()

    def make_body(g_hbm):
      def body(j, _):
        pltpu.async_copy(g_hbm.at[src_v.at[j]], rows_v, sem).wait()
        pltpu.sync_copy(rows_v, acc_s.at[dst_v.at[j]], add=True)
        return 0
      return body

    @pl.when(cid == 0)
    def _():
      lax.fori_loop(0, chunks_per_tile, make_body(glo_hbm), 0)

    @pl.when(cid == 1)
    def _():
      lax.fori_loop(0, chunks_per_tile, make_body(ghi_hbm), 0)

    plsc.subcore_barrier()

    @pl.when(cid == 0)
    def _():
      pltpu.sync_copy(acc_s.at[pl.ds(sid * slab, slab)],
                      out_lo.at[pl.ds(sid * slab, slab)])

    @pl.when(cid == 1)
    def _():
      pltpu.sync_copy(acc_s.at[pl.ds(sid * slab, slab)],
                      out_hi.at[pl.ds(sid * slab, slab)])

  return agg_kernel


def _make_sc_agg_split(nrows, chunks_per_tile, width):
  @functools.partial(
      pl.kernel,
      out_type=(jax.ShapeDtypeStruct((nrows, width), jnp.float32),
                jax.ShapeDtypeStruct((nrows, width), jnp.float32)),
      mesh=_sc_mesh(),
      scratch_types=[
          pltpu.VMEM((chunks_per_tile, CHUNK), jnp.int32),
          pltpu.VMEM((chunks_per_tile, CHUNK), jnp.int32),
          pltpu.VMEM((CHUNK, width), jnp.float32),
          pltpu.VMEM_SHARED((nrows, width), jnp.float32),
          pltpu.SemaphoreType.DMA,
      ],
  )
  def agg_kernel(src_hbm, dst_hbm, g_hbm, out0, out1,
                 src_v, dst_v, rows_v, acc_s, sem):
    cid = lax.axis_index("c")
    sid = lax.axis_index("s")
    wid = cid * NS + sid
    slab = nrows // NS

    pltpu.sync_copy(src_hbm.at[pl.ds(wid * chunks_per_tile, chunks_per_tile)],
                    src_v)
    pltpu.sync_copy(dst_hbm.at[pl.ds(wid * chunks_per_tile, chunks_per_tile)],
                    dst_v)
    _zero_fill(rows_v, width)
    _zero_slab(rows_v, acc_s, sid * slab, slab)
    plsc.subcore_barrier()

    def body(j, _):
      pltpu.async_copy(g_hbm.at[src_v.at[j]], rows_v, sem).wait()
      pltpu.sync_copy(rows_v, acc_s.at[dst_v.at[j]], add=True)
      return 0

    lax.fori_loop(0, chunks_per_tile, body, 0)
    plsc.subcore_barrier()

    @pl.when(cid == 0)
    def _():
      pltpu.sync_copy(acc_s.at[pl.ds(sid * slab, slab)],
                      out0.at[pl.ds(sid * slab, slab)])

    @pl.when(cid == 1)
    def _():
      pltpu.sync_copy(acc_s.at[pl.ds(sid * slab, slab)],
                      out1.at[pl.ds(sid * slab, slab)])

  return agg_kernel


def _dinv_of(d0_ref, d1_ref):
  deg = d0_ref[:, 0:1] + d1_ref[:, 0:1]
  return lax.rsqrt(deg)


def _make_tc_first(n, k, m):
  grid = n // ROW_BLOCK
  mh = m // 2

  def body(x_ref, w_ref, d0_ref, d1_ref, glo_ref, ghi_ref):
    dinv = _dinv_of(d0_ref, d1_ref)
    h = jnp.dot(x_ref[...], w_ref[...], preferred_element_type=jnp.float32)
    g = h * dinv
    glo_ref[...] = g[:, :mh]
    ghi_ref[...] = g[:, mh:]

  return pl.pallas_call(
      body,
      grid=(grid,),
      in_specs=[
          pl.BlockSpec((ROW_BLOCK, k), lambda i: (i, 0)),
          pl.BlockSpec((k, m), lambda i: (0, 0)),
          pl.BlockSpec((ROW_BLOCK, 16), lambda i: (i, 0)),
          pl.BlockSpec((ROW_BLOCK, 16), lambda i: (i, 0)),
      ],
      out_specs=[
          pl.BlockSpec((ROW_BLOCK, mh), lambda i: (i, 0)),
          pl.BlockSpec((ROW_BLOCK, mh), lambda i: (i, 0)),
      ],
      out_shape=[
          jax.ShapeDtypeStruct((n, mh), jnp.float32),
          jax.ShapeDtypeStruct((n, mh), jnp.float32),
      ],
  )


def _make_tc_mid_pad(n, k, m, mpad):
  grid = n // ROW_BLOCK
  kh = k // 2

  def body(alo_ref, ahi_ref, glo_ref, ghi_ref, d0_ref, d1_ref, b_ref, w_ref,
           o_ref):
    dinv = _dinv_of(d0_ref, d1_ref)
    t_lo = alo_ref[...] + glo_ref[...]
    t_hi = ahi_ref[...] + ghi_ref[...]
    t = jnp.concatenate([t_lo, t_hi], axis=1) * dinv + b_ref[...]
    t = jnp.maximum(t, 0.0)
    h = jnp.dot(t, w_ref[...], preferred_element_type=jnp.float32)
    g = h * dinv
    o_ref[...] = jnp.pad(g, ((0, 0), (0, mpad - m)))

  return pl.pallas_call(
      body,
      grid=(grid,),
      in_specs=[
          pl.BlockSpec((ROW_BLOCK, kh), lambda i: (i, 0)),
          pl.BlockSpec((ROW_BLOCK, kh), lambda i: (i, 0)),
          pl.BlockSpec((ROW_BLOCK, kh), lambda i: (i, 0)),
          pl.BlockSpec((ROW_BLOCK, kh), lambda i: (i, 0)),
          pl.BlockSpec((ROW_BLOCK, 16), lambda i: (i, 0)),
          pl.BlockSpec((ROW_BLOCK, 16), lambda i: (i, 0)),
          pl.BlockSpec((1, k), lambda i: (0, 0)),
          pl.BlockSpec((k, m), lambda i: (0, 0)),
      ],
      out_specs=pl.BlockSpec((ROW_BLOCK, mpad), lambda i: (i, 0)),
      out_shape=jax.ShapeDtypeStruct((n, mpad), jnp.float32),
  )


def _make_tc_mid_unpad(n, k, kpad, m):
  grid = n // ROW_BLOCK
  mh = m // 2

  def body(a0_ref, a1_ref, gp_ref, d0_ref, d1_ref, b_ref, w_ref,
           olo_ref, ohi_ref):
    dinv = _dinv_of(d0_ref, d1_ref)
    t = (a0_ref[...] + a1_ref[...] + gp_ref[...])[:, :k] * dinv + b_ref[...]
    h = jnp.dot(t, w_ref[...], preferred_element_type=jnp.float32)
    g = h * dinv
    olo_ref[...] = g[:, :mh]
    ohi_ref[...] = g[:, mh:]

  return pl.pallas_call(
      body,
      grid=(grid,),
      in_specs=[
          pl.BlockSpec((ROW_BLOCK, kpad), lambda i: (i, 0)),
          pl.BlockSpec((ROW_BLOCK, kpad), lambda i: (i, 0)),
          pl.BlockSpec((ROW_BLOCK, kpad), lambda i: (i, 0)),
          pl.BlockSpec((ROW_BLOCK, 16), lambda i: (i, 0)),
          pl.BlockSpec((ROW_BLOCK, 16), lambda i: (i, 0)),
          pl.BlockSpec((1, k), lambda i: (0, 0)),
          pl.BlockSpec((k, m), lambda i: (0, 0)),
      ],
      out_specs=[
          pl.BlockSpec((ROW_BLOCK, mh), lambda i: (i, 0)),
          pl.BlockSpec((ROW_BLOCK, mh), lambda i: (i, 0)),
      ],
      out_shape=[
          jax.ShapeDtypeStruct((n, mh), jnp.float32),
          jax.ShapeDtypeStruct((n, mh), jnp.float32),
      ],
  )


def _make_tc_mid(n, nrows, k, m, relu):
  grid = n // ROW_BLOCK
  kh = k // 2
  mh = m // 2

  def body(alo_ref, ahi_ref, glo_ref, ghi_ref, d0_ref, d1_ref, b_ref, w_ref,
           olo_ref, ohi_ref):
    dinv = _dinv_of(d0_ref, d1_ref)
    t_lo = (alo_ref[...] + glo_ref[...])
    t_hi = (ahi_ref[...] + ghi_ref[...])
    t = jnp.concatenate([t_lo, t_hi], axis=1) * dinv + b_ref[...]
    if relu:
      t = jnp.maximum(t, 0.0)
    h = jnp.dot(t, w_ref[...], preferred_element_type=jnp.float32)
    g = h * dinv
    olo_ref[...] = g[:, :mh]
    ohi_ref[...] = g[:, mh:]

  return pl.pallas_call(
      body,
      grid=(grid,),
      in_specs=[
          pl.BlockSpec((ROW_BLOCK, kh), lambda i: (i, 0)),
          pl.BlockSpec((ROW_BLOCK, kh), lambda i: (i, 0)),
          pl.BlockSpec((ROW_BLOCK, kh), lambda i: (i, 0)),
          pl.BlockSpec((ROW_BLOCK, kh), lambda i: (i, 0)),
          pl.BlockSpec((ROW_BLOCK, 16), lambda i: (i, 0)),
          pl.BlockSpec((ROW_BLOCK, 16), lambda i: (i, 0)),
          pl.BlockSpec((1, k), lambda i: (0, 0)),
          pl.BlockSpec((k, m), lambda i: (0, 0)),
      ],
      out_specs=[
          pl.BlockSpec((ROW_BLOCK, mh), lambda i: (i, 0)),
          pl.BlockSpec((ROW_BLOCK, mh), lambda i: (i, 0)),
      ],
      out_shape=[
          jax.ShapeDtypeStruct((n, mh), jnp.float32),
          jax.ShapeDtypeStruct((n, mh), jnp.float32),
      ],
  )


def _make_tc_final(n, nrows, k):
  grid = n // ROW_BLOCK
  kh = k // 2

  def body(alo_ref, ahi_ref, glo_ref, ghi_ref, d0_ref, d1_ref, b_ref,
           out_ref):
    dinv = _dinv_of(d0_ref, d1_ref)
    t_lo = alo_ref[...] + glo_ref[...]
    t_hi = ahi_ref[...] + ghi_ref[...]
    t = jnp.concatenate([t_lo, t_hi], axis=1) * dinv + b_ref[...]
    out_ref[...] = t

  return pl.pallas_call(
      body,
      grid=(grid,),
      in_specs=[
          pl.BlockSpec((ROW_BLOCK, kh), lambda i: (i, 0)),
          pl.BlockSpec((ROW_BLOCK, kh), lambda i: (i, 0)),
          pl.BlockSpec((ROW_BLOCK, kh), lambda i: (i, 0)),
          pl.BlockSpec((ROW_BLOCK, kh), lambda i: (i, 0)),
          pl.BlockSpec((ROW_BLOCK, 16), lambda i: (i, 0)),
          pl.BlockSpec((ROW_BLOCK, 16), lambda i: (i, 0)),
          pl.BlockSpec((1, k), lambda i: (0, 0)),
      ],
      out_specs=pl.BlockSpec((ROW_BLOCK, k), lambda i: (i, 0)),
      out_shape=jax.ShapeDtypeStruct((n, k), jnp.float32),
  )


@jax.jit
def kernel(x, edge_index, W1, b1, W2, b2, W3, b3, W4, b4):
  n, in_dim = x.shape
  hid = W1.shape[1]
  emb = W2.shape[1]
  e = edge_index.shape[1]

  nrows = ((n + 1 + NS * 8 - 1) // (NS * 8)) * (NS * 8)
  chunks = ((e + CHUNK - 1) // CHUNK + 2 * NS - 1) // (2 * NS) * (2 * NS)
  e_pad = chunks * CHUNK
  cpt_deg = chunks // (2 * NS)
  cpt_agg = chunks // NS

  src = edge_index[0]
  dst = edge_index[1]
  pad = e_pad - e
  src_p = jnp.concatenate([src, jnp.zeros((pad,), jnp.int32)])
  dst_p = jnp.concatenate([dst, jnp.full((pad,), n, jnp.int32)])
  src_p = src_p.reshape(chunks, CHUNK)
  dst_p = dst_p.reshape(chunks, CHUNK)

  d0, d1 = _make_sc_deg(nrows, cpt_deg)(dst_p)
  d0 = d0 + 1.0

  agg_wide = _make_sc_agg(nrows, cpt_agg, hid // 2)
  agg_pad = _make_sc_agg_split(nrows, cpt_deg, 128)

  b1r = b1.reshape(1, -1)
  b2r = b2.reshape(1, -1)
  b3r = b3.reshape(1, -1)
  b4r = b4.reshape(1, -1)

  g1_lo, g1_hi = _make_tc_first(n, in_dim, hid)(x, W1, d0, d1)
  a1_lo, a1_hi = agg_wide(src_p, dst_p, g1_lo, g1_hi)
  g2p = _make_tc_mid_pad(n, hid, emb, 128)(
      a1_lo, a1_hi, g1_lo, g1_hi, d0, d1, b1r, W2)
  a2_0, a2_1 = agg_pad(src_p, dst_p, g2p)
  g3_lo, g3_hi = _make_tc_mid_unpad(n, emb, 128, hid)(
      a2_0, a2_1, g2p, d0, d1, b2r, W3)
  a3_lo, a3_hi = agg_wide(src_p, dst_p, g3_lo, g3_hi)
  g4_lo, g4_hi = _make_tc_mid(n, nrows, hid, in_dim, True)(
      a3_lo, a3_hi, g3_lo, g3_hi, d0, d1, b3r, W4)
  a4_lo, a4_hi = agg_wide(src_p, dst_p, g4_lo, g4_hi)
  out = _make_tc_final(n, nrows, in_dim)(
      a4_lo, a4_hi, g4_lo, g4_hi, d0, d1, b4r)
  return out


# speedup vs baseline: 6.1284x; 1.0042x over previous
import functools

import jax
import jax.numpy as jnp
from jax import lax
from jax.experimental import pallas as pl
from jax.experimental.pallas import tpu as pltpu
from jax.experimental.pallas import tpu_sc as plsc

NC = 2
NS = 16
CHUNK = 128
NBUF = 2
IDXSTRIP = 40
ROW_BLOCK = 2000


def _sc_mesh():
  return plsc.VectorSubcoreMesh(
      core_axis_name="c", subcore_axis_name="s", num_cores=NC,
      num_subcores=NS)


def _zero_fill(ref, width):
  zeros = jnp.zeros((16,), jnp.float32)

  def body(i, _):
    for w in range(width // 16):
      ref[i, pl.ds(w * 16, 16)] = zeros
    return 0

  lax.fori_loop(0, CHUNK, body, 0)


def _ones_fill(ref):
  ones = jnp.ones((16,), jnp.float32)

  def body(i, _):
    ref[i] = ones
    return 0

  lax.fori_loop(0, CHUNK, body, 0)


def _zero_slab(zero_v, acc_s, row0, rows):
  full, rem = divmod(rows, CHUNK)
  for i in range(full):
    pltpu.sync_copy(zero_v, acc_s.at[pl.ds(row0 + i * CHUNK, CHUNK)])
  if rem:
    pltpu.sync_copy(zero_v.at[pl.ds(0, rem)],
                    acc_s.at[pl.ds(row0 + full * CHUNK, rem)])


def _make_sc_deg(nrows, chunks_per_tile):
  @functools.partial(
      pl.kernel,
      out_type=(jax.ShapeDtypeStruct((nrows, 16), jnp.float32),
                jax.ShapeDtypeStruct((nrows, 16), jnp.float32)),
      mesh=_sc_mesh(),
      scratch_types=[
          pltpu.VMEM((chunks_per_tile, CHUNK), jnp.int32),
          pltpu.VMEM((CHUNK, 16), jnp.float32),
          pltpu.VMEM((CHUNK, 16), jnp.float32),
          pltpu.VMEM_SHARED((nrows, 16), jnp.float32),
      ],
  )
  def deg_kernel(dst_hbm, out0, out1, dst_v, ones_v, zero_v, acc_s):
    cid = lax.axis_index("c")
    sid = lax.axis_index("s")
    wid = cid * NS + sid
    slab = nrows // NS

    pltpu.sync_copy(dst_hbm.at[pl.ds(wid * chunks_per_tile, chunks_per_tile)],
                    dst_v)
    _ones_fill(ones_v)
    _zero_fill(zero_v, 16)
    _zero_slab(zero_v, acc_s, sid * slab, slab)
    plsc.subcore_barrier()

    def body(j, _):
      pltpu.sync_copy(ones_v, acc_s.at[dst_v.at[j]], add=True)
      return 0

    lax.fori_loop(0, chunks_per_tile, body, 0)
    plsc.subcore_barrier()

    @pl.when(cid == 0)
    def _():
      pltpu.sync_copy(acc_s.at[pl.ds(sid * slab, slab)],
                      out0.at[pl.ds(sid * slab, slab)])

    @pl.when(cid == 1)
    def _():
      pltpu.sync_copy(acc_s.at[pl.ds(sid * slab, slab)],
                      out1.at[pl.ds(sid * slab, slab)])

  return deg_kernel


def _ring_agg(g_hbm, src_hbm, dst_hbm, c0, cpt, src_v, dst_v, bufs, sems,
              acc_s):
  for p in range(cpt // IDXSTRIP):
    base = c0 + p * IDXSTRIP
    pltpu.sync_copy(src_hbm.at[pl.ds(base, IDXSTRIP)], src_v)
    pltpu.sync_copy(dst_hbm.at[pl.ds(base, IDXSTRIP)], dst_v)

    def outer(i, _):
      j0 = i * NBUF
      cps = [pltpu.async_copy(g_hbm.at[src_v.at[j0 + b]], bufs[b], sems[b])
             for b in range(NBUF)]
      for b in range(NBUF):
        cps[b].wait()
        pltpu.sync_copy(bufs[b], acc_s.at[dst_v.at[j0 + b]], add=True)
      return 0

    lax.fori_loop(0, IDXSTRIP // NBUF, outer, 0)


def _make_sc_agg(nrows, chunks_per_tile, width):
  @functools.partial(
      pl.kernel,
      out_type=(jax.ShapeDtypeStruct((nrows, width), jnp.float32),
                jax.ShapeDtypeStruct((nrows, width), jnp.float32)),
      mesh=_sc_mesh(),
      scratch_types=[
          pltpu.VMEM((chunks_per_tile, CHUNK), jnp.int32),
          pltpu.VMEM((chunks_per_tile, CHUNK), jnp.int32),
          pltpu.VMEM((CHUNK, width), jnp.float32),
          pltpu.VMEM_SHARED((nrows, width), jnp.float32),
          pltpu.SemaphoreType.DMA,
      ],
  )
  def agg_kernel(src_hbm, dst_hbm, glo_hbm, ghi_hbm, out_lo, out_hi,
                 src_v, dst_v, rows_v, acc_s, sem):
    cid = lax.axis_index("c")
    sid = lax.axis_index("s")
    slab = nrows // NS

    pltpu.sync_copy(src_hbm.at[pl.ds(sid * chunks_per_tile, chunks_per_tile)],
                    src_v)
    pltpu.sync_copy(dst_hbm.at[pl.ds(sid * chunks_per_tile, chunks_per_tile)],
                    dst_v)
    _zero_fill(rows_v, width)
    _zero_slab(rows_v, acc_s, sid * slab, slab)
    plsc.subcore_barrier()

    def make_body(g_hbm):
      def body(j, _):
        pltpu.async_copy(g_hbm.at[src_v.at[j]], rows_v, sem).wait()
        pltpu.sync_copy(rows_v, acc_s.at[dst_v.at[j]], add=True)
        return 0
      return body

    @pl.when(cid == 0)
    def _():
      lax.fori_loop(0, chunks_per_tile, make_body(glo_hbm), 0)

    @pl.when(cid == 1)
    def _():
      lax.fori_loop(0, chunks_per_tile, make_body(ghi_hbm), 0)

    plsc.subcore_barrier()

    @pl.when(cid == 0)
    def _():
      pltpu.sync_copy(acc_s.at[pl.ds(sid * slab, slab)],
                      out_lo.at[pl.ds(sid * slab, slab)])

    @pl.when(cid == 1)
    def _():
      pltpu.sync_copy(acc_s.at[pl.ds(sid * slab, slab)],
                      out_hi.at[pl.ds(sid * slab, slab)])

  return agg_kernel


def _make_sc_agg_split(nrows, chunks_per_tile, width):
  @functools.partial(
      pl.kernel,
      out_type=(jax.ShapeDtypeStruct((nrows, width), jnp.float32),
                jax.ShapeDtypeStruct((nrows, width), jnp.float32)),
      mesh=_sc_mesh(),
      scratch_types=[
          pltpu.VMEM((chunks_per_tile, CHUNK), jnp.int32),
          pltpu.VMEM((chunks_per_tile, CHUNK), jnp.int32),
          pltpu.VMEM((CHUNK, width), jnp.float32),
          pltpu.VMEM((CHUNK, width), jnp.float32),
          pltpu.VMEM_SHARED((nrows, width), jnp.float32),
          pltpu.SemaphoreType.DMA,
          pltpu.SemaphoreType.DMA,
      ],
  )
  def agg_kernel(src_hbm, dst_hbm, g_hbm, out0, out1,
                 src_v, dst_v, r0, r1, acc_s, s0, s1):
    cid = lax.axis_index("c")
    sid = lax.axis_index("s")
    wid = cid * NS + sid
    slab = nrows // NS

    pltpu.sync_copy(src_hbm.at[pl.ds(wid * chunks_per_tile, chunks_per_tile)],
                    src_v)
    pltpu.sync_copy(dst_hbm.at[pl.ds(wid * chunks_per_tile, chunks_per_tile)],
                    dst_v)
    _zero_fill(r0, width)
    _zero_slab(r0, acc_s, sid * slab, slab)
    plsc.subcore_barrier()

    def body(i, _):
      j0 = i * 2
      cp0 = pltpu.async_copy(g_hbm.at[src_v.at[j0]], r0, s0)
      cp1 = pltpu.async_copy(g_hbm.at[src_v.at[j0 + 1]], r1, s1)
      cp0.wait()
      pltpu.sync_copy(r0, acc_s.at[dst_v.at[j0]], add=True)
      cp1.wait()
      pltpu.sync_copy(r1, acc_s.at[dst_v.at[j0 + 1]], add=True)
      return 0

    lax.fori_loop(0, chunks_per_tile // 2, body, 0)
    plsc.subcore_barrier()

    @pl.when(cid == 0)
    def _():
      pltpu.sync_copy(acc_s.at[pl.ds(sid * slab, slab)],
                      out0.at[pl.ds(sid * slab, slab)])

    @pl.when(cid == 1)
    def _():
      pltpu.sync_copy(acc_s.at[pl.ds(sid * slab, slab)],
                      out1.at[pl.ds(sid * slab, slab)])

  return agg_kernel


def _dinv_of(d0_ref, d1_ref):
  deg = d0_ref[:, 0:1] + d1_ref[:, 0:1]
  return lax.rsqrt(deg)


def _make_tc_first(n, k, m):
  grid = n // ROW_BLOCK
  mh = m // 2

  def body(x_ref, w_ref, d0_ref, d1_ref, glo_ref, ghi_ref):
    dinv = _dinv_of(d0_ref, d1_ref)
    h = jnp.dot(x_ref[...], w_ref[...], preferred_element_type=jnp.float32)
    g = h * dinv
    glo_ref[...] = g[:, :mh]
    ghi_ref[...] = g[:, mh:]

  return pl.pallas_call(
      body,
      grid=(grid,),
      in_specs=[
          pl.BlockSpec((ROW_BLOCK, k), lambda i: (i, 0)),
          pl.BlockSpec((k, m), lambda i: (0, 0)),
          pl.BlockSpec((ROW_BLOCK, 16), lambda i: (i, 0)),
          pl.BlockSpec((ROW_BLOCK, 16), lambda i: (i, 0)),
      ],
      out_specs=[
          pl.BlockSpec((ROW_BLOCK, mh), lambda i: (i, 0)),
          pl.BlockSpec((ROW_BLOCK, mh), lambda i: (i, 0)),
      ],
      out_shape=[
          jax.ShapeDtypeStruct((n, mh), jnp.float32),
          jax.ShapeDtypeStruct((n, mh), jnp.float32),
      ],
  )


def _make_tc_mid_pad(n, k, m, mpad):
  grid = n // ROW_BLOCK
  kh = k // 2

  def body(alo_ref, ahi_ref, glo_ref, ghi_ref, d0_ref, d1_ref, b_ref, w_ref,
           o_ref):
    dinv = _dinv_of(d0_ref, d1_ref)
    t_lo = alo_ref[...] + glo_ref[...]
    t_hi = ahi_ref[...] + ghi_ref[...]
    t = jnp.concatenate([t_lo, t_hi], axis=1) * dinv + b_ref[...]
    t = jnp.maximum(t, 0.0)
    h = jnp.dot(t, w_ref[...], preferred_element_type=jnp.float32)
    g = h * dinv
    o_ref[...] = jnp.pad(g, ((0, 0), (0, mpad - m)))

  return pl.pallas_call(
      body,
      grid=(grid,),
      in_specs=[
          pl.BlockSpec((ROW_BLOCK, kh), lambda i: (i, 0)),
          pl.BlockSpec((ROW_BLOCK, kh), lambda i: (i, 0)),
          pl.BlockSpec((ROW_BLOCK, kh), lambda i: (i, 0)),
          pl.BlockSpec((ROW_BLOCK, kh), lambda i: (i, 0)),
          pl.BlockSpec((ROW_BLOCK, 16), lambda i: (i, 0)),
          pl.BlockSpec((ROW_BLOCK, 16), lambda i: (i, 0)),
          pl.BlockSpec((1, k), lambda i: (0, 0)),
          pl.BlockSpec((k, m), lambda i: (0, 0)),
      ],
      out_specs=pl.BlockSpec((ROW_BLOCK, mpad), lambda i: (i, 0)),
      out_shape=jax.ShapeDtypeStruct((n, mpad), jnp.float32),
  )


def _make_tc_mid_unpad(n, k, kpad, m):
  grid = n // ROW_BLOCK
  mh = m // 2

  def body(a0_ref, a1_ref, gp_ref, d0_ref, d1_ref, b_ref, w_ref,
           olo_ref, ohi_ref):
    dinv = _dinv_of(d0_ref, d1_ref)
    t = (a0_ref[...] + a1_ref[...] + gp_ref[...])[:, :k] * dinv + b_ref[...]
    h = jnp.dot(t, w_ref[...], preferred_element_type=jnp.float32)
    g = h * dinv
    olo_ref[...] = g[:, :mh]
    ohi_ref[...] = g[:, mh:]

  return pl.pallas_call(
      body,
      grid=(grid,),
      in_specs=[
          pl.BlockSpec((ROW_BLOCK, kpad), lambda i: (i, 0)),
          pl.BlockSpec((ROW_BLOCK, kpad), lambda i: (i, 0)),
          pl.BlockSpec((ROW_BLOCK, kpad), lambda i: (i, 0)),
          pl.BlockSpec((ROW_BLOCK, 16), lambda i: (i, 0)),
          pl.BlockSpec((ROW_BLOCK, 16), lambda i: (i, 0)),
          pl.BlockSpec((1, k), lambda i: (0, 0)),
          pl.BlockSpec((k, m), lambda i: (0, 0)),
      ],
      out_specs=[
          pl.BlockSpec((ROW_BLOCK, mh), lambda i: (i, 0)),
          pl.BlockSpec((ROW_BLOCK, mh), lambda i: (i, 0)),
      ],
      out_shape=[
          jax.ShapeDtypeStruct((n, mh), jnp.float32),
          jax.ShapeDtypeStruct((n, mh), jnp.float32),
      ],
  )


def _make_tc_mid(n, nrows, k, m, relu):
  grid = n // ROW_BLOCK
  kh = k // 2
  mh = m // 2

  def body(alo_ref, ahi_ref, glo_ref, ghi_ref, d0_ref, d1_ref, b_ref, w_ref,
           olo_ref, ohi_ref):
    dinv = _dinv_of(d0_ref, d1_ref)
    t_lo = (alo_ref[...] + glo_ref[...])
    t_hi = (ahi_ref[...] + ghi_ref[...])
    t = jnp.concatenate([t_lo, t_hi], axis=1) * dinv + b_ref[...]
    if relu:
      t = jnp.maximum(t, 0.0)
    h = jnp.dot(t, w_ref[...], preferred_element_type=jnp.float32)
    g = h * dinv
    olo_ref[...] = g[:, :mh]
    ohi_ref[...] = g[:, mh:]

  return pl.pallas_call(
      body,
      grid=(grid,),
      in_specs=[
          pl.BlockSpec((ROW_BLOCK, kh), lambda i: (i, 0)),
          pl.BlockSpec((ROW_BLOCK, kh), lambda i: (i, 0)),
          pl.BlockSpec((ROW_BLOCK, kh), lambda i: (i, 0)),
          pl.BlockSpec((ROW_BLOCK, kh), lambda i: (i, 0)),
          pl.BlockSpec((ROW_BLOCK, 16), lambda i: (i, 0)),
          pl.BlockSpec((ROW_BLOCK, 16), lambda i: (i, 0)),
          pl.BlockSpec((1, k), lambda i: (0, 0)),
          pl.BlockSpec((k, m), lambda i: (0, 0)),
      ],
      out_specs=[
          pl.BlockSpec((ROW_BLOCK, mh), lambda i: (i, 0)),
          pl.BlockSpec((ROW_BLOCK, mh), lambda i: (i, 0)),
      ],
      out_shape=[
          jax.ShapeDtypeStruct((n, mh), jnp.float32),
          jax.ShapeDtypeStruct((n, mh), jnp.float32),
      ],
  )


def _make_tc_final(n, nrows, k):
  grid = n // ROW_BLOCK
  kh = k // 2

  def body(alo_ref, ahi_ref, glo_ref, ghi_ref, d0_ref, d1_ref, b_ref,
           out_ref):
    dinv = _dinv_of(d0_ref, d1_ref)
    t_lo = alo_ref[...] + glo_ref[...]
    t_hi = ahi_ref[...] + ghi_ref[...]
    t = jnp.concatenate([t_lo, t_hi], axis=1) * dinv + b_ref[...]
    out_ref[...] = t

  return pl.pallas_call(
      body,
      grid=(grid,),
      in_specs=[
          pl.BlockSpec((ROW_BLOCK, kh), lambda i: (i, 0)),
          pl.BlockSpec((ROW_BLOCK, kh), lambda i: (i, 0)),
          pl.BlockSpec((ROW_BLOCK, kh), lambda i: (i, 0)),
          pl.BlockSpec((ROW_BLOCK, kh), lambda i: (i, 0)),
          pl.BlockSpec((ROW_BLOCK, 16), lambda i: (i, 0)),
          pl.BlockSpec((ROW_BLOCK, 16), lambda i: (i, 0)),
          pl.BlockSpec((1, k), lambda i: (0, 0)),
      ],
      out_specs=pl.BlockSpec((ROW_BLOCK, k), lambda i: (i, 0)),
      out_shape=jax.ShapeDtypeStruct((n, k), jnp.float32),
  )


@jax.jit
def kernel(x, edge_index, W1, b1, W2, b2, W3, b3, W4, b4):
  n, in_dim = x.shape
  hid = W1.shape[1]
  emb = W2.shape[1]
  e = edge_index.shape[1]

  nrows = ((n + 1 + NS * 8 - 1) // (NS * 8)) * (NS * 8)
  cdiv = 2 * NS * IDXSTRIP
  chunks = ((e + CHUNK - 1) // CHUNK + cdiv - 1) // cdiv * cdiv
  e_pad = chunks * CHUNK
  cpt_deg = chunks // (2 * NS)
  cpt_agg = chunks // NS

  src = edge_index[0]
  dst = edge_index[1]
  pad = e_pad - e
  src_p = jnp.concatenate([src, jnp.zeros((pad,), jnp.int32)])
  dst_p = jnp.concatenate([dst, jnp.full((pad,), n, jnp.int32)])
  src_p = src_p.reshape(chunks, CHUNK)
  dst_p = dst_p.reshape(chunks, CHUNK)

  d0, d1 = _make_sc_deg(nrows, cpt_deg)(dst_p)
  d0 = d0 + 1.0

  agg_wide = _make_sc_agg(nrows, cpt_agg, hid // 2)
  agg_pad = _make_sc_agg_split(nrows, cpt_deg, 128)

  b1r = b1.reshape(1, -1)
  b2r = b2.reshape(1, -1)
  b3r = b3.reshape(1, -1)
  b4r = b4.reshape(1, -1)

  g1_lo, g1_hi = _make_tc_first(n, in_dim, hid)(x, W1, d0, d1)
  a1_lo, a1_hi = agg_wide(src_p, dst_p, g1_lo, g1_hi)
  g2p = _make_tc_mid_pad(n, hid, emb, 128)(
      a1_lo, a1_hi, g1_lo, g1_hi, d0, d1, b1r, W2)
  a2_0, a2_1 = agg_pad(src_p, dst_p, g2p)
  g3_lo, g3_hi = _make_tc_mid_unpad(n, emb, 128, hid)(
      a2_0, a2_1, g2p, d0, d1, b2r, W3)
  a3_lo, a3_hi = agg_wide(src_p, dst_p, g3_lo, g3_hi)
  g4_lo, g4_hi = _make_tc_mid(n, nrows, hid, in_dim, True)(
      a3_lo, a3_hi, g3_lo, g3_hi, d0, d1, b3r, W4)
  a4_lo, a4_hi = agg_wide(src_p, dst_p, g4_lo, g4_hi)
  out = _make_tc_final(n, nrows, in_dim)(
      a4_lo, a4_hi, g4_lo, g4_hi, d0, d1, b4r)
  return out
